# Initial kernel scaffold; baseline (speedup 1.0000x reference)
import jax
import jax.numpy as jnp
from jax.experimental import pallas as pl


def kernel(for_gen, R_fake, t_fake, r_buffer, t_buffer):
    raise NotImplementedError("write your pallas kernel here")


# speedup vs baseline: 2.6549x; 2.6549x over previous
import functools

import jax
import jax.numpy as jnp
import numpy as np
from jax.experimental import pallas as pl
from jax.experimental.pallas import tpu as pltpu

NR = 8192
NF = 2048
BN = 1024

_EPS_T = 1e-05
_CLIP = 1e-06


def _f32_matmul(x, y):
    xh = x.astype(jnp.bfloat16)
    xl = (x - xh.astype(jnp.float32)).astype(jnp.bfloat16)
    yh = y.astype(jnp.bfloat16)
    yl = (y - yh.astype(jnp.float32)).astype(jnp.bfloat16)
    dn = (((1,), (0,)), ((), ()))
    mm = functools.partial(jax.lax.dot_general, dimension_numbers=dn,
                           preferred_element_type=jnp.float32)
    return mm(xh, yh) + (mm(xh, yl) + mm(xl, yh))


def _acos(x):
    return jnp.arctan2(jnp.sqrt((1.0 + x) * (1.0 - x)), x)


def _chamfer_body(rb_ref, rft_ref, tb_ref, tft_ref, out_ref,
                  colmin_ref, colmax_ref, acc_ref):
    i = pl.program_id(0)
    nblk = pl.num_programs(0)

    tr = _f32_matmul(rb_ref[...], rft_ref[...])

    tb = tb_ref[...]
    tft = tft_ref[...]
    ab = jax.lax.dot_general(
        tb.astype(jnp.bfloat16), tft.astype(jnp.bfloat16),
        (((1,), (0,)), ((), ())), preferred_element_type=jnp.float32)
    a2 = jnp.sum(tb * tb, axis=1, keepdims=True)
    b2 = jnp.sum(tft * tft, axis=0, keepdims=True)
    d = jnp.maximum(a2 + b2 - 2.0 * ab, 0.0)

    row_min_d = jnp.min(d, axis=1, keepdims=True)
    row_max_tr = jnp.max(tr, axis=1, keepdims=True)
    part_sqrt = jnp.sum(jnp.sqrt(row_min_d + _EPS_T))
    c_row = jnp.clip((row_max_tr - 1.0) * 0.5, -1.0 + _CLIP, 1.0 - _CLIP)
    part_acos = jnp.sum(_acos(c_row))

    cmin_d = jnp.min(d, axis=0, keepdims=True)
    cmax_tr = jnp.max(tr, axis=0, keepdims=True)

    @pl.when(i == 0)
    def _():
        colmin_ref[...] = cmin_d
        colmax_ref[...] = cmax_tr
        acc_ref[0] = part_sqrt
        acc_ref[1] = part_acos

    @pl.when(i > 0)
    def _():
        colmin_ref[...] = jnp.minimum(colmin_ref[...], cmin_d)
        colmax_ref[...] = jnp.maximum(colmax_ref[...], cmax_tr)
        acc_ref[0] = acc_ref[0] + part_sqrt
        acc_ref[1] = acc_ref[1] + part_acos

    @pl.when(i == nblk - 1)
    def _():
        td1_mean = acc_ref[0] / NR
        rd2_mean = acc_ref[1] / NR
        td2_mean = jnp.mean(jnp.sqrt(colmin_ref[...] + _EPS_T))
        c_col = jnp.clip((colmax_ref[...] - 1.0) * 0.5,
                         -1.0 + _CLIP, 1.0 - _CLIP)
        rd1_mean = jnp.mean(_acos(c_col))
        tloss = td1_mean + td2_mean
        rloss = rd1_mean + rd2_mean
        loss = rloss + tloss * np.float32(np.pi)
        out_ref[...] = jnp.broadcast_to(loss, (1, 1))


@functools.partial(jax.jit, static_argnames=())
def _chamfer_loss(rb_flat, rft, tb, tft):
    grid = NR // BN
    out = pl.pallas_call(
        _chamfer_body,
        grid=(grid,),
        in_specs=[
            pl.BlockSpec((BN, 9), lambda i: (i, 0)),
            pl.BlockSpec((9, NF), lambda i: (0, 0)),
            pl.BlockSpec((BN, 3), lambda i: (i, 0)),
            pl.BlockSpec((3, NF), lambda i: (0, 0)),
        ],
        out_specs=pl.BlockSpec((1, 1), lambda i: (0, 0)),
        out_shape=jax.ShapeDtypeStruct((1, 1), jnp.float32),
        scratch_shapes=[
            pltpu.VMEM((1, NF), jnp.float32),
            pltpu.VMEM((1, NF), jnp.float32),
            pltpu.SMEM((2,), jnp.float32),
        ],
    )(rb_flat, rft, tb, tft)
    return out[0, 0]


def kernel(for_gen, R_fake, t_fake, r_buffer, t_buffer):
    rb_flat = r_buffer.reshape(NR, 9)
    rft = R_fake.reshape(NF, 9).T
    tb = t_buffer[0]
    tft = t_fake.T
    loss = _chamfer_loss(rb_flat, rft, tb, tft)
    return jnp.where(for_gen != 0, loss, jnp.zeros((), dtype=jnp.float32))
